# Initial kernel scaffold; baseline (speedup 1.0000x reference)
import jax
import jax.numpy as jnp
from jax.experimental import pallas as pl


def kernel(x, edge_index, edge_type, batch, enc_W, enc_b, edge_emb, c1_linW, c1_linb, c1_W1, c1_b1, c1_g, c1_be, c1_W2, c1_b2, c2_linW, c2_linb, c2_W1, c2_b1, c2_g, c2_be, c2_W2, c2_b2, head_W, head_b, clf_W, clf_b):
    raise NotImplementedError("write your pallas kernel here")


# speedup vs baseline: 3.3728x; 3.3728x over previous
import functools

import jax
import jax.numpy as jnp
from jax import lax
from jax.experimental import pallas as pl
from jax.experimental.pallas import tpu as pltpu
from jax.experimental.pallas import tpu_sc as plsc

EPS = 1e-5
NC = 2
NS = 16
L = 16

_HI = lax.Precision.HIGHEST


def _dot(a, b):
    return jnp.dot(a, b, precision=_HI, preferred_element_type=jnp.float32)


def _pre_body(x_ref, encW_ref, encb_ref, emb_ref, l1W_ref, l1b_ref,
              l2W_ref, l2b_ref, h_ref, t1_ref, t2_ref):
    h_ref[...] = _dot(x_ref[...], encW_ref[...]) + encb_ref[...]
    emb = emb_ref[...]
    t1_ref[...] = _dot(emb, l1W_ref[...]) + l1b_ref[...]
    t2_ref[...] = _dot(emb, l2W_ref[...]) + l2b_ref[...]


def _mlp(u, W1, b1, g, be, W2, b2):
    v = _dot(u, W1) + b1
    m = jnp.mean(v, axis=0, keepdims=True)
    var = jnp.mean((v - m) ** 2, axis=0, keepdims=True)
    v = g * (v - m) / jnp.sqrt(var + EPS) + be
    v = jnp.maximum(v, 0.0)
    return _dot(v, W2) + b2


def _mid_body(h_ref, p_ref, W1_ref, b1_ref, g_ref, be_ref, W2_ref, b2_ref,
              o_ref):
    u = h_ref[...] + p_ref[0, :, :] + p_ref[1, :, :]
    w = _mlp(u, W1_ref[...], b1_ref[...], g_ref[...], be_ref[...],
             W2_ref[...], b2_ref[...])
    o_ref[...] = jnp.maximum(w, 0.0)


def _final_body(h_ref, p_ref, W1_ref, b1_ref, g_ref, be_ref, W2_ref, b2_ref,
                batch_ref, headW_ref, headb_ref, clfW_ref, clfb_ref, o_ref):
    u = h_ref[...] + p_ref[0, :, :] + p_ref[1, :, :]
    h2 = _mlp(u, W1_ref[...], b1_ref[...], g_ref[...], be_ref[...],
              W2_ref[...], b2_ref[...])
    n, _ = h2.shape
    G = o_ref.shape[0]
    onehot = (batch_ref[...] == lax.broadcasted_iota(jnp.int32, (n, G), 1))
    pooled = _dot(onehot.astype(jnp.float32).T, h2)
    z = jnp.maximum(_dot(pooled, headW_ref[...]) + headb_ref[...], 0.0)
    o_ref[...] = _dot(z, clfW_ref[...]) + clfb_ref[...]


def _edge_pass(h, src, dst, etype, t):
    N, H = h.shape
    E = src.shape[0]
    W = NC * NS
    assert E % W == 0
    epw = E // W
    C = 80
    assert epw % C == 0
    nch = epw // C
    rpa = (N // NS) // 8 * 8
    rem = N - rpa * NS
    assert rem % 8 == 0 and rem >= 0
    zrows = rpa + rem
    nfb = H // L

    mesh = plsc.VectorSubcoreMesh(core_axis_name="c", subcore_axis_name="s")

    @functools.partial(
        pl.kernel,
        out_type=jax.ShapeDtypeStruct((NC, N, H), jnp.float32),
        mesh=mesh,
        scratch_types=[
            pltpu.VMEM((C,), jnp.int32),
            pltpu.VMEM((C,), jnp.int32),
            pltpu.VMEM((C,), jnp.int32),
            pltpu.VMEM((C, H), jnp.float32),
            pltpu.VMEM((t.shape[0] * H,), jnp.float32),
            pltpu.VMEM((zrows, H), jnp.float32),
            pltpu.VMEM_SHARED((N, H), jnp.float32),
            pltpu.SemaphoreType.DMA,
        ],
        compiler_params=pltpu.CompilerParams(use_tc_tiling_on_sc=False),
    )
    def k(h_hbm, src_hbm, dst_hbm, typ_hbm, t_hbm, out_hbm,
          sidx, didx, typv, rows, tloc, zbuf, aggr, sem):
        cid = lax.axis_index("c")
        sid = lax.axis_index("s")
        wid = cid * NS + sid

        zero = jnp.zeros((L,), jnp.float32)

        def zrow(i, carry):
            r = i // nfb
            col = (i % nfb) * L
            zbuf[r, pl.ds(col, L)] = zero
            return carry

        lax.fori_loop(0, zrows * nfb, zrow, 0)
        pltpu.sync_copy(zbuf.at[pl.ds(0, rpa)], aggr.at[pl.ds(sid * rpa, rpa)])

        @pl.when(sid == NS - 1)
        def _():
            pltpu.sync_copy(zbuf.at[pl.ds(rpa, rem)],
                            aggr.at[pl.ds(NS * rpa, rem)])

        pltpu.sync_copy(t_hbm, tloc)
        plsc.subcore_barrier()

        base_w = wid * epw

        def chunk(ci, carry):
            base = base_w + ci * C
            pltpu.sync_copy(src_hbm.at[pl.ds(base, C)], sidx)
            pltpu.sync_copy(dst_hbm.at[pl.ds(base, C)], didx)
            pltpu.sync_copy(typ_hbm.at[pl.ds(base, C)], typv)
            pltpu.async_copy(h_hbm.at[sidx], rows, sem).wait()

            def group_body(g, c2):
                tv16 = typv[pl.ds(g * L, L)] * H
                e0 = g * L
                for i in range(L):
                    toff = tv16[i]
                    for j in range(nfb):
                        hv = rows[e0 + i, pl.ds(j * L, L)]
                        tv = tloc[pl.ds(toff + j * L, L)]
                        rows[e0 + i, pl.ds(j * L, L)] = jnp.maximum(hv + tv, 0.0)
                return c2

            lax.fori_loop(0, C // L, group_body, 0)
            pltpu.sync_copy(rows, aggr.at[didx], add=True)
            return carry

        lax.fori_loop(0, nch, chunk, 0)

        plsc.subcore_barrier()
        pltpu.sync_copy(aggr.at[pl.ds(sid * rpa, rpa)], zbuf.at[pl.ds(0, rpa)])
        pltpu.sync_copy(zbuf.at[pl.ds(0, rpa)],
                        out_hbm.at[cid, pl.ds(sid * rpa, rpa)])

        @pl.when(sid == NS - 1)
        def _():
            pltpu.sync_copy(aggr.at[pl.ds(NS * rpa, rem)],
                            zbuf.at[pl.ds(rpa, rem)])
            pltpu.sync_copy(zbuf.at[pl.ds(rpa, rem)],
                            out_hbm.at[cid, pl.ds(NS * rpa, rem)])

    return k(h, src, dst, etype, t.reshape(-1))


def kernel(x, edge_index, edge_type, batch, enc_W, enc_b, edge_emb,
           c1_linW, c1_linb, c1_W1, c1_b1, c1_g, c1_be, c1_W2, c1_b2,
           c2_linW, c2_linb, c2_W1, c2_b1, c2_g, c2_be, c2_W2, c2_b2,
           head_W, head_b, clf_W, clf_b):
    N, _ = x.shape
    H = enc_W.shape[1]
    G = 128
    OUT = clf_W.shape[1]
    src = edge_index[0]
    dst = edge_index[1]

    row = lambda v: v.reshape(1, -1)

    h0, t1, t2 = pl.pallas_call(
        _pre_body,
        out_shape=(
            jax.ShapeDtypeStruct((N, H), jnp.float32),
            jax.ShapeDtypeStruct((edge_emb.shape[0], H), jnp.float32),
            jax.ShapeDtypeStruct((edge_emb.shape[0], H), jnp.float32),
        ),
    )(x, enc_W, row(enc_b), edge_emb, c1_linW, row(c1_linb), c2_linW,
      row(c2_linb))

    p1 = _edge_pass(h0, src, dst, edge_type, t1)

    h1 = pl.pallas_call(
        _mid_body,
        out_shape=jax.ShapeDtypeStruct((N, H), jnp.float32),
    )(h0, p1, c1_W1, row(c1_b1), row(c1_g), row(c1_be), c1_W2, row(c1_b2))

    p2 = _edge_pass(h1, src, dst, edge_type, t2)

    out = pl.pallas_call(
        _final_body,
        out_shape=jax.ShapeDtypeStruct((G, OUT), jnp.float32),
    )(h1, p2, c2_W1, row(c2_b1), row(c2_g), row(c2_be), c2_W2, row(c2_b2),
      batch.reshape(-1, 1), head_W, row(head_b), clf_W, row(clf_b))

    return out


# speedup vs baseline: 6.1991x; 1.8380x over previous
import functools

import jax
import jax.numpy as jnp
from jax import lax
from jax.experimental import pallas as pl
from jax.experimental.pallas import tpu as pltpu
from jax.experimental.pallas import tpu_sc as plsc

EPS = 1e-5
NC = 2
NS = 16
L = 16

_HI = lax.Precision.HIGHEST


def _dot(a, b):
    return jnp.dot(a, b, precision=_HI, preferred_element_type=jnp.float32)


def _pre_body(x_ref, encW_ref, encb_ref, emb_ref, l1W_ref, l1b_ref,
              l2W_ref, l2b_ref, h_ref, t1_ref, t2_ref):
    h_ref[...] = _dot(x_ref[...], encW_ref[...]) + encb_ref[...]
    emb = emb_ref[...]
    t1_ref[...] = _dot(emb, l1W_ref[...]) + l1b_ref[...]
    t2_ref[...] = _dot(emb, l2W_ref[...]) + l2b_ref[...]


def _mlp(u, W1, b1, g, be, W2, b2):
    v = _dot(u, W1) + b1
    m = jnp.mean(v, axis=0, keepdims=True)
    var = jnp.mean((v - m) ** 2, axis=0, keepdims=True)
    v = g * (v - m) / jnp.sqrt(var + EPS) + be
    v = jnp.maximum(v, 0.0)
    return _dot(v, W2) + b2


def _mid_body(h_ref, p_ref, W1_ref, b1_ref, g_ref, be_ref, W2_ref, b2_ref,
              o_ref):
    u = h_ref[...] + p_ref[0, :, :] + p_ref[1, :, :]
    w = _mlp(u, W1_ref[...], b1_ref[...], g_ref[...], be_ref[...],
             W2_ref[...], b2_ref[...])
    o_ref[...] = jnp.maximum(w, 0.0)


def _final_body(h_ref, p_ref, W1_ref, b1_ref, g_ref, be_ref, W2_ref, b2_ref,
                batch_ref, headW_ref, headb_ref, clfW_ref, clfb_ref, o_ref):
    u = h_ref[...] + p_ref[0, :, :] + p_ref[1, :, :]
    h2 = _mlp(u, W1_ref[...], b1_ref[...], g_ref[...], be_ref[...],
              W2_ref[...], b2_ref[...])
    n, _ = h2.shape
    G = o_ref.shape[0]
    onehot = (batch_ref[...] == lax.broadcasted_iota(jnp.int32, (n, G), 1))
    pooled = _dot(onehot.astype(jnp.float32).T, h2)
    z = jnp.maximum(_dot(pooled, headW_ref[...]) + headb_ref[...], 0.0)
    o_ref[...] = _dot(z, clfW_ref[...]) + clfb_ref[...]


def _edge_pass(h, src, dst, etype, t):
    N, H = h.shape
    E = src.shape[0]
    R = t.shape[0]
    W = NC * NS
    assert E % W == 0
    epw = E // W
    C = 80
    assert epw % C == 0
    nch = epw // C
    npair = (nch - 1) // 2
    assert nch == 2 * npair + 1
    rpa = (N // NS) // 8 * 8
    rem = N - rpa * NS
    assert rem % 8 == 0 and rem >= 0
    zrows = rpa + rem
    nfb = H // L

    mesh = plsc.VectorSubcoreMesh(core_axis_name="c", subcore_axis_name="s")

    @functools.partial(
        pl.kernel,
        out_type=jax.ShapeDtypeStruct((NC, N, H), jnp.float32),
        mesh=mesh,
        scratch_types=[
            pltpu.VMEM((2, C), jnp.int32),
            pltpu.VMEM((2, C), jnp.int32),
            pltpu.VMEM((2, C), jnp.int32),
            pltpu.VMEM((2, C, H), jnp.float32),
            pltpu.VMEM((2, C, H), jnp.float32),
            pltpu.VMEM((zrows, H), jnp.float32),
            pltpu.VMEM((R, H), jnp.float32),
            pltpu.VMEM_SHARED((N, H), jnp.float32),
            pltpu.VMEM_SHARED((R, H), jnp.float32),
            pltpu.SemaphoreType.DMA,
            pltpu.SemaphoreType.DMA,
            pltpu.SemaphoreType.DMA,
            pltpu.SemaphoreType.DMA,
            pltpu.SemaphoreType.DMA,
            pltpu.SemaphoreType.DMA,
        ],
        compiler_params=pltpu.CompilerParams(use_tc_tiling_on_sc=False),
    )
    def k(h_hbm, src_hbm, dst_hbm, typ_hbm, t_hbm, out_hbm,
          sidx, didx, tidx, rows, trows, zbuf, tvm, aggr, tspm,
          semg0, semg1, semt0, semt1, sems0, sems1):
        cid = lax.axis_index("c")
        sid = lax.axis_index("s")
        wid = cid * NS + sid
        semg = (semg0, semg1)
        semt = (semt0, semt1)
        sems = (sems0, sems1)

        @pl.when(sid == 0)
        def _():
            pltpu.sync_copy(t_hbm, tvm)
            pltpu.sync_copy(tvm, tspm)

        zero = jnp.zeros((L,), jnp.float32)

        def zrow(i, carry):
            r = i // nfb
            col = (i % nfb) * L
            zbuf[r, pl.ds(col, L)] = zero
            return carry

        lax.fori_loop(0, zrows * nfb, zrow, 0)
        pltpu.sync_copy(zbuf.at[pl.ds(0, rpa)], aggr.at[pl.ds(sid * rpa, rpa)])

        @pl.when(sid == NS - 1)
        def _():
            pltpu.sync_copy(zbuf.at[pl.ds(rpa, rem)],
                            aggr.at[pl.ds(NS * rpa, rem)])

        plsc.subcore_barrier()

        base_w = wid * epw

        def fetch(c, b):
            base = base_w + c * C
            pltpu.sync_copy(src_hbm.at[pl.ds(base, C)], sidx.at[b])
            pltpu.sync_copy(dst_hbm.at[pl.ds(base, C)], didx.at[b])
            pltpu.sync_copy(typ_hbm.at[pl.ds(base, C)], tidx.at[b])
            pltpu.async_copy(h_hbm.at[sidx.at[b]], rows.at[b], semg[b])
            pltpu.async_copy(tspm.at[tidx.at[b]], trows.at[b], semt[b])

        def wait_gather(b):
            pltpu.make_async_copy(h_hbm.at[sidx.at[b]], rows.at[b],
                                  semg[b]).wait()
            pltpu.make_async_copy(tspm.at[tidx.at[b]], trows.at[b],
                                  semt[b]).wait()

        def compute(b):
            @plsc.parallel_loop(0, C, unroll=2)
            def _(r):
                for j in range(nfb):
                    s = pl.ds(j * L, L)
                    rows[b, r, s] = jnp.maximum(rows[b, r, s] + trows[b, r, s],
                                                0.0)

        def scatter(b):
            pltpu.async_copy(rows.at[b], aggr.at[didx.at[b]], sems[b],
                             add=True)

        def wait_scatter(b):
            pltpu.make_async_copy(rows.at[b], aggr.at[didx.at[b]],
                                  sems[b]).wait()

        fetch(0, 0)

        def pair(kk, carry):
            @pl.when(kk > 0)
            def _():
                wait_scatter(1)

            fetch(2 * kk + 1, 1)
            wait_gather(0)
            compute(0)
            scatter(0)
            wait_gather(1)
            compute(1)
            scatter(1)
            wait_scatter(0)
            fetch(2 * kk + 2, 0)
            return carry

        lax.fori_loop(0, npair, pair, 0)
        wait_scatter(1)
        wait_gather(0)
        compute(0)
        scatter(0)
        wait_scatter(0)

        plsc.subcore_barrier()
        pltpu.sync_copy(aggr.at[pl.ds(sid * rpa, rpa)], zbuf.at[pl.ds(0, rpa)])
        pltpu.sync_copy(zbuf.at[pl.ds(0, rpa)],
                        out_hbm.at[cid, pl.ds(sid * rpa, rpa)])

        @pl.when(sid == NS - 1)
        def _():
            pltpu.sync_copy(aggr.at[pl.ds(NS * rpa, rem)],
                            zbuf.at[pl.ds(rpa, rem)])
            pltpu.sync_copy(zbuf.at[pl.ds(rpa, rem)],
                            out_hbm.at[cid, pl.ds(NS * rpa, rem)])

    return k(h, src, dst, etype, t)


def kernel(x, edge_index, edge_type, batch, enc_W, enc_b, edge_emb,
           c1_linW, c1_linb, c1_W1, c1_b1, c1_g, c1_be, c1_W2, c1_b2,
           c2_linW, c2_linb, c2_W1, c2_b1, c2_g, c2_be, c2_W2, c2_b2,
           head_W, head_b, clf_W, clf_b):
    N, _ = x.shape
    H = enc_W.shape[1]
    G = 128
    OUT = clf_W.shape[1]
    src = edge_index[0]
    dst = edge_index[1]

    row = lambda v: v.reshape(1, -1)

    h0, t1, t2 = pl.pallas_call(
        _pre_body,
        out_shape=(
            jax.ShapeDtypeStruct((N, H), jnp.float32),
            jax.ShapeDtypeStruct((edge_emb.shape[0], H), jnp.float32),
            jax.ShapeDtypeStruct((edge_emb.shape[0], H), jnp.float32),
        ),
    )(x, enc_W, row(enc_b), edge_emb, c1_linW, row(c1_linb), c2_linW,
      row(c2_linb))

    p1 = _edge_pass(h0, src, dst, edge_type, t1)

    h1 = pl.pallas_call(
        _mid_body,
        out_shape=jax.ShapeDtypeStruct((N, H), jnp.float32),
    )(h0, p1, c1_W1, row(c1_b1), row(c1_g), row(c1_be), c1_W2, row(c1_b2))

    p2 = _edge_pass(h1, src, dst, edge_type, t2)

    out = pl.pallas_call(
        _final_body,
        out_shape=jax.ShapeDtypeStruct((G, OUT), jnp.float32),
    )(h1, p2, c2_W1, row(c2_b1), row(c2_g), row(c2_be), c2_W2, row(c2_b2),
      batch.reshape(-1, 1), head_W, row(head_b), clf_W, row(clf_b))

    return out


# speedup vs baseline: 12.9263x; 2.0852x over previous
import functools

import jax
import jax.numpy as jnp
from jax import lax
from jax.experimental import pallas as pl
from jax.experimental.pallas import tpu as pltpu
from jax.experimental.pallas import tpu_sc as plsc

EPS = 1e-5
NC = 2
NS = 16
L = 16
EC = 80
PD = 5
PF = 3

_HI = lax.Precision.HIGHEST


def _dot(a, b):
    return jnp.dot(a, b, precision=_HI, preferred_element_type=jnp.float32)


def _pre_body(x_ref, encW_ref, encb_ref, emb_ref, l1W_ref, l1b_ref,
              l2W_ref, l2b_ref, h_ref, t1_ref, t2_ref):
    h_ref[...] = _dot(x_ref[...], encW_ref[...]) + encb_ref[...]
    emb = emb_ref[...]
    t1_ref[...] = _dot(emb, l1W_ref[...]) + l1b_ref[...]
    t2_ref[...] = _dot(emb, l2W_ref[...]) + l2b_ref[...]


def _mlp(u, W1, b1, g, be, W2, b2):
    v = _dot(u, W1) + b1
    m = jnp.mean(v, axis=0, keepdims=True)
    var = jnp.mean((v - m) ** 2, axis=0, keepdims=True)
    v = g * (v - m) / jnp.sqrt(var + EPS) + be
    v = jnp.maximum(v, 0.0)
    return _dot(v, W2) + b2


def _mid_body(h_ref, p_ref, W1_ref, b1_ref, g_ref, be_ref, W2_ref, b2_ref,
              o_ref):
    u = h_ref[...] + p_ref[0, :, :] + p_ref[1, :, :]
    w = _mlp(u, W1_ref[...], b1_ref[...], g_ref[...], be_ref[...],
             W2_ref[...], b2_ref[...])
    o_ref[...] = jnp.maximum(w, 0.0)


def _final_body(h_ref, p_ref, W1_ref, b1_ref, g_ref, be_ref, W2_ref, b2_ref,
                batch_ref, headW_ref, headb_ref, clfW_ref, clfb_ref, o_ref):
    u = h_ref[...] + p_ref[0, :, :] + p_ref[1, :, :]
    h2 = _mlp(u, W1_ref[...], b1_ref[...], g_ref[...], be_ref[...],
              W2_ref[...], b2_ref[...])
    n, _ = h2.shape
    G = o_ref.shape[0]
    onehot = (batch_ref[...] == lax.broadcasted_iota(jnp.int32, (n, G), 1))
    pooled = _dot(onehot.astype(jnp.float32).T, h2)
    z = jnp.maximum(_dot(pooled, headW_ref[...]) + headb_ref[...], 0.0)
    o_ref[...] = _dot(z, clfW_ref[...]) + clfb_ref[...]


def _edge_pass(h, src2, dst2, typ2, t):
    N, H = h.shape
    R = t.shape[0]
    nrows_all = src2.shape[0]
    W = NC * NS
    assert nrows_all % W == 0
    nch = nrows_all // W
    assert nch % PD == 0
    niter = nch // PD
    nfb = H // L
    rpa = (N // NS) // 8 * 8
    rem = N - rpa * NS
    assert rem % 8 == 0 and rem <= EC
    nzc = rpa // EC
    zrem = rpa - nzc * EC

    mesh = plsc.VectorSubcoreMesh(core_axis_name="c", subcore_axis_name="s")

    @functools.partial(
        pl.kernel,
        out_type=jax.ShapeDtypeStruct((NC, N, H), jnp.float32),
        mesh=mesh,
        scratch_types=[
            pltpu.VMEM((nch, EC), jnp.int32),
            pltpu.VMEM((nch, EC), jnp.int32),
            pltpu.VMEM((nch, EC), jnp.int32),
            pltpu.VMEM((PD, EC, H), jnp.float32),
            pltpu.VMEM((PD, EC, H), jnp.float32),
            pltpu.VMEM((R, H), jnp.float32),
            pltpu.VMEM_SHARED((N, H), jnp.float32),
            pltpu.VMEM_SHARED((R, H), jnp.float32),
            pltpu.SemaphoreType.DMA((PD,)),
            pltpu.SemaphoreType.DMA((PD,)),
            pltpu.SemaphoreType.DMA((PD,)),
        ],
        compiler_params=pltpu.CompilerParams(use_tc_tiling_on_sc=False),
    )
    def k(h_hbm, src_hbm, dst_hbm, typ_hbm, t_hbm, out_hbm,
          sidxall, didxall, tidxall, rows, trows, tvm, aggr, tspm,
          semg, semt, sems):
        cid = lax.axis_index("c")
        sid = lax.axis_index("s")
        wid = cid * NS + sid
        crow0 = wid * nch

        @pl.when(sid == 0)
        def _():
            pltpu.sync_copy(t_hbm, tvm)
            pltpu.sync_copy(tvm, tspm)

        pltpu.sync_copy(src_hbm.at[pl.ds(crow0, nch)], sidxall)
        pltpu.sync_copy(dst_hbm.at[pl.ds(crow0, nch)], didxall)
        pltpu.sync_copy(typ_hbm.at[pl.ds(crow0, nch)], tidxall)

        zero = jnp.zeros((L,), jnp.float32)

        def zrow(i, carry):
            r = i // nfb
            col = (i % nfb) * L
            trows[0, r, pl.ds(col, L)] = zero
            return carry

        lax.fori_loop(0, EC * nfb, zrow, 0)
        for i in range(nzc):
            pltpu.sync_copy(trows.at[0],
                            aggr.at[pl.ds(sid * rpa + i * EC, EC)])
        if zrem:
            pltpu.sync_copy(trows.at[0, pl.ds(0, zrem)],
                            aggr.at[pl.ds(sid * rpa + nzc * EC, zrem)])

        @pl.when(sid == NS - 1)
        def _():
            pltpu.sync_copy(trows.at[0, pl.ds(0, rem)],
                            aggr.at[pl.ds(NS * rpa, rem)])

        plsc.subcore_barrier()

        def fire(c, b):
            pltpu.async_copy(h_hbm.at[sidxall.at[c]], rows.at[b],
                             semg.at[b])
            pltpu.async_copy(tspm.at[tidxall.at[c]], trows.at[b],
                             semt.at[b])

        def wait_gather(c, b):
            pltpu.make_async_copy(h_hbm.at[sidxall.at[c]], rows.at[b],
                                  semg.at[b]).wait()
            pltpu.make_async_copy(tspm.at[tidxall.at[c]], trows.at[b],
                                  semt.at[b]).wait()

        def scatter(c, b):
            pltpu.async_copy(rows.at[b], aggr.at[didxall.at[c]], sems.at[b],
                             add=True)

        def wait_scatter(c, b):
            pltpu.make_async_copy(rows.at[b], aggr.at[didxall.at[c]],
                                  sems.at[b]).wait()

        def compute(b):
            @plsc.parallel_loop(0, EC, unroll=2)
            def _(r):
                for j in range(nfb):
                    s = pl.ds(j * L, L)
                    rows[b, r, s] = jnp.maximum(rows[b, r, s] + trows[b, r, s],
                                                0.0)

        for c0 in range(PF):
            fire(c0, c0)

        def step(kk, carry):
            cbase = kk * PD
            for b in range(PD):
                c = cbase + b
                cf = c + PF
                bf = (b + PF) % PD

                @pl.when(cf < nch)
                def _():
                    @pl.when(cf >= PD)
                    def _():
                        wait_scatter(cf - PD, bf)

                    fire(cf, bf)

                wait_gather(c, b)
                compute(b)
                scatter(c, b)
            return carry

        lax.fori_loop(0, niter, step, 0)
        for b in range(PD):
            wait_scatter(nch - PD + b, b)

        plsc.subcore_barrier()
        pltpu.sync_copy(aggr.at[pl.ds(sid * rpa, rpa)],
                        out_hbm.at[cid, pl.ds(sid * rpa, rpa)])

        @pl.when(sid == NS - 1)
        def _():
            pltpu.sync_copy(aggr.at[pl.ds(NS * rpa, rem)],
                            out_hbm.at[cid, pl.ds(NS * rpa, rem)])

    return k(h, src2, dst2, typ2, t)


def kernel(x, edge_index, edge_type, batch, enc_W, enc_b, edge_emb,
           c1_linW, c1_linb, c1_W1, c1_b1, c1_g, c1_be, c1_W2, c1_b2,
           c2_linW, c2_linb, c2_W1, c2_b1, c2_g, c2_be, c2_W2, c2_b2,
           head_W, head_b, clf_W, clf_b):
    N, _ = x.shape
    H = enc_W.shape[1]
    G = 128
    OUT = clf_W.shape[1]
    src2 = edge_index[0].reshape(-1, EC)
    dst2 = edge_index[1].reshape(-1, EC)
    typ2 = edge_type.reshape(-1, EC)

    row = lambda v: v.reshape(1, -1)

    h0, t1, t2 = pl.pallas_call(
        _pre_body,
        out_shape=(
            jax.ShapeDtypeStruct((N, H), jnp.float32),
            jax.ShapeDtypeStruct((edge_emb.shape[0], H), jnp.float32),
            jax.ShapeDtypeStruct((edge_emb.shape[0], H), jnp.float32),
        ),
    )(x, enc_W, row(enc_b), edge_emb, c1_linW, row(c1_linb), c2_linW,
      row(c2_linb))

    p1 = _edge_pass(h0, src2, dst2, typ2, t1)

    h1 = pl.pallas_call(
        _mid_body,
        out_shape=jax.ShapeDtypeStruct((N, H), jnp.float32),
    )(h0, p1, c1_W1, row(c1_b1), row(c1_g), row(c1_be), c1_W2, row(c1_b2))

    p2 = _edge_pass(h1, src2, dst2, typ2, t2)

    out = pl.pallas_call(
        _final_body,
        out_shape=jax.ShapeDtypeStruct((G, OUT), jnp.float32),
    )(h1, p2, c2_W1, row(c2_b1), row(c2_g), row(c2_be), c2_W2, row(c2_b2),
      batch.reshape(-1, 1), head_W, row(head_b), clf_W, row(clf_b))

    return out
